# baseline (speedup 1.0000x reference)
import functools

import jax
import jax.numpy as jnp
from jax import lax
from jax.experimental import pallas as pl
from jax.experimental.pallas import tpu as pltpu
from jax.experimental.pallas import tpu_sc as plsc

BEAM = 4
VOCAB = 100000
PAD = 1
ROWS = 512
BSZ = ROWS // BEAM
K2 = 2 * BEAM

NC, NS, L = 2, 16, 16
NW = NC * NS
ROWS_PER_W = ROWS // NW
SLICES = VOCAB // L
GROUP = 5
ITERS = SLICES // GROUP

NEG = float("-inf")


def _sc_body(logits_hbm, stat_hbm, idx_hbm, rowbuf, vstage, istage):
    wid = lax.axis_index("s") * NC + lax.axis_index("c")
    lane = lax.iota(jnp.int32, L)

    def do_row(r, _):
        row = wid * ROWS_PER_W + r
        pltpu.sync_copy(logits_hbm.at[row], rowbuf)

        def a_body(i, m):
            base = i * (GROUP * L)
            xs = [rowbuf[pl.ds(base + j * L, L)] for j in range(GROUP)]
            t01 = jnp.maximum(xs[0], xs[1])
            t23 = jnp.maximum(xs[2], xs[3])
            return jnp.maximum(m, jnp.maximum(jnp.maximum(t01, t23), xs[4]))

        m = lax.fori_loop(0, ITERS, a_body, jnp.full((L,), NEG, jnp.float32))

        def b_body(i, carry):
            s0, s1, s2, s3, s4, V, I, thr = carry
            base = i * (GROUP * L)
            xs = [rowbuf[pl.ds(base + j * L, L)] for j in range(GROUP)]
            s0 = s0 + jnp.exp(xs[0] - m)
            s1 = s1 + jnp.exp(xs[1] - m)
            s2 = s2 + jnp.exp(xs[2] - m)
            s3 = s3 + jnp.exp(xs[3] - m)
            s4 = s4 + jnp.exp(xs[4] - m)
            t01 = jnp.maximum(xs[0], xs[1])
            t23 = jnp.maximum(xs[2], xs[3])
            mx = jnp.maximum(jnp.maximum(t01, t23), xs[4])
            hit = jnp.any(mx > thr)

            def merge(ops):
                V, I, _ = ops
                for j in range(GROUP):
                    idxv = base + j * L + lane
                    x = jnp.where(idxv == PAD, NEG, xs[j])
                    xd, xid = plsc.sort_key_val(x, idxv, descending=True)
                    keep = xd > V
                    V2 = jnp.where(keep, xd, V)
                    I2 = jnp.where(keep, xid, I)
                    V, I = plsc.sort_key_val(V2, I2, descending=False)
                thrv = V.at[jnp.zeros((L,), jnp.int32)].get(
                    mode="promise_in_bounds")
                return V, I, thrv

            V, I, thr = lax.cond(hit, merge, lambda ops: ops, (V, I, thr))
            return (s0, s1, s2, s3, s4, V, I, thr)

        z = jnp.zeros((L,), jnp.float32)
        init = (z, z, z, z, z,
                jnp.full((L,), NEG, jnp.float32),
                jnp.zeros((L,), jnp.int32),
                jnp.full((L,), NEG, jnp.float32))
        s0, s1, s2, s3, s4, V, I, _ = lax.fori_loop(0, ITERS, b_body, init)
        s = (s0 + s1) + (s2 + s3) + s4

        vstage[0] = m
        vstage[1] = s
        vstage[2] = V
        istage[...] = I
        pltpu.sync_copy(vstage, stat_hbm.at[row])
        pltpu.sync_copy(istage, idx_hbm.at[row])
        return 0

    lax.fori_loop(0, ROWS_PER_W, do_row, 0)


_sc_topk = functools.partial(
    pl.kernel,
    out_type=[
        jax.ShapeDtypeStruct((ROWS, 3, L), jnp.float32),
        jax.ShapeDtypeStruct((ROWS, L), jnp.int32),
    ],
    mesh=plsc.VectorSubcoreMesh(
        core_axis_name="c", subcore_axis_name="s",
        num_cores=NC, num_subcores=NS),
    scratch_types=[
        pltpu.VMEM((VOCAB,), jnp.float32),
        pltpu.VMEM((3, L), jnp.float32),
        pltpu.VMEM((L,), jnp.int32),
    ],
    compiler_params=pltpu.CompilerParams(needs_layout_passes=False),
)(_sc_body)


def _tc_merge_body(m_ref, s_ref, v_ref, i_ref, sc_ref, os_ref, ot_ref, ob_ref):
    m = m_ref[...]
    s = s_ref[...]
    V = v_ref[...]
    I = i_ref[...]
    sc = sc_ref[...]

    seg = lax.broadcasted_iota(jnp.int32, (BSZ, BEAM * L), 1) // L
    offs = jnp.zeros((BSZ, BEAM * L), jnp.float32)
    for j in range(BEAM):
        maskj = seg == j
        mj = jnp.max(jnp.where(maskj, m, NEG), axis=1, keepdims=True)
        sj = jnp.sum(jnp.where(maskj, s * jnp.exp(m - mj), 0.0),
                     axis=1, keepdims=True)
        offj = sc[:, j:j + 1] - (mj + jnp.log(sj))
        offs = jnp.where(maskj, offj, offs)

    cand = V + offs
    flat = I + seg * VOCAB
    big = jnp.int32(2**31 - 1)
    vals, flats = [], []
    for _ in range(K2):
        cur = jnp.max(cand, axis=1, keepdims=True)
        cf = jnp.min(jnp.where(cand == cur, flat, big), axis=1, keepdims=True)
        vals.append(cur)
        flats.append(cf)
        cand = jnp.where(flat == cf, NEG, cand)
    ts = jnp.concatenate(vals, axis=1)
    tf = jnp.concatenate(flats, axis=1)
    os_ref[...] = ts
    ot_ref[...] = tf % VOCAB
    ob_ref[...] = tf // VOCAB


_tc_merge = pl.pallas_call(
    _tc_merge_body,
    out_shape=(
        jax.ShapeDtypeStruct((BSZ, K2), jnp.float32),
        jax.ShapeDtypeStruct((BSZ, K2), jnp.int32),
        jax.ShapeDtypeStruct((BSZ, K2), jnp.int32),
    ),
)


def kernel(logits, scores):
    stat, idx = _sc_topk(logits)
    m4 = stat[:, 0, :].reshape(BSZ, BEAM * L)
    s4 = stat[:, 1, :].reshape(BSZ, BEAM * L)
    v4 = stat[:, 2, :].reshape(BSZ, BEAM * L)
    i4 = idx.reshape(BSZ, BEAM * L)
    sc4 = scores.reshape(BSZ, BEAM)
    return _tc_merge(m4, s4, v4, i4, sc4)


# speedup vs baseline: 3.9256x; 3.9256x over previous
import functools

import jax
import jax.numpy as jnp
from jax import lax
from jax.experimental import pallas as pl
from jax.experimental.pallas import tpu as pltpu
from jax.experimental.pallas import tpu_sc as plsc

BEAM = 4
VOCAB = 100000
PAD = 1
ROWS = 512
BSZ = ROWS // BEAM
K2 = 2 * BEAM

NC, NS, L = 2, 16, 16
NW = NC * NS
ROWS_PER_W = ROWS // NW
SLICES = VOCAB // L
GROUP = 5
ITERS = SLICES // GROUP

NEG = float("-inf")


CAPJ = 32
NCAND = GROUP * CAPJ * L
GROUP_A = 10
ITERS_A = SLICES // GROUP_A


def _merge_slice(x, iv, V, I):
    xd, xid = plsc.sort_key_val(x, iv, descending=True)
    keep = xd > V
    V2 = jnp.where(keep, xd, V)
    I2 = jnp.where(keep, xid, I)
    Vn, In = plsc.sort_key_val(V2, I2, descending=False)
    return (Vn, In)


def _sc_body(logits_hbm, stat_hbm, idx_hbm, rowbuf, candi, vstage, istage):
    wid = lax.axis_index("s") * NC + lax.axis_index("c")
    lane = lax.iota(jnp.int32, L)

    def zi(j, _):
        candi[pl.ds(j * L, L)] = jnp.zeros((L,), jnp.int32)
        return 0

    lax.fori_loop(0, NCAND // L, zi, 0)

    def do_row(r, _):
        row = wid * ROWS_PER_W + r
        pltpu.sync_copy(logits_hbm.at[row], rowbuf)

        neg = jnp.full((L,), NEG, jnp.float32)

        def a_body(base, ms):
            ma, mb = ms
            xs = [rowbuf[pl.ds(base + j * L, L)] for j in range(GROUP_A)]
            ta = jnp.maximum(jnp.maximum(xs[0], xs[1]),
                             jnp.maximum(xs[2], xs[3]))
            tb = jnp.maximum(jnp.maximum(xs[5], xs[6]),
                             jnp.maximum(xs[7], xs[8]))
            ta = jnp.maximum(ta, xs[4])
            tb = jnp.maximum(tb, xs[9])
            return (jnp.maximum(ma, ta), jnp.maximum(mb, tb))

        ma, mb = plsc.parallel_loop(
            0, VOCAB, step=GROUP_A * L, unroll=2, carry=(neg, neg))(a_body)
        m = jnp.maximum(ma, mb)

        t = m
        for k in (8, 4, 2, 1):
            perm = (lane + k) & (L - 1)
            t = jnp.minimum(t, t.at[perm].get(mode="promise_in_bounds"))
        thr = t

        wrap = CAPJ * L - 1

        def b_body(base, carry):
            sacc = list(carry[:GROUP])
            cnts = list(carry[GROUP:])
            for j in range(GROUP):
                x = rowbuf[pl.ds(base + j * L, L)]
                sacc[j] = sacc[j] + jnp.exp(x - m)
                hitm = x >= thr
                idxv = base + j * L + lane
                pos = (cnts[j] & wrap) + j * (CAPJ * L)
                plsc.store_scatter(candi, [pos], idxv, mask=hitm)
                cnts[j] = cnts[j] + jnp.where(hitm, L, 0)
            return (*sacc, *cnts)

        z = jnp.zeros((L,), jnp.float32)
        out = plsc.parallel_loop(
            0, VOCAB, step=GROUP * L, unroll=2,
            carry=(z,) * GROUP + (lane,) * GROUP)(b_body)
        sacc = out[:GROUP]
        cnts = out[GROUP:]
        s = (sacc[0] + sacc[1]) + (sacc[2] + sacc[3]) + sacc[4]

        cmax = cnts[0]
        for j in range(1, GROUP):
            cmax = jnp.maximum(cmax, cnts[j])
        overflow = jnp.any(cmax - lane > wrap)

        def fast(ops):
            V, I = ops
            for j in range(GROUP):
                nslots = jnp.max(cnts[j] - lane) // L
                rb = j * (CAPJ * L)

                def mbody(q, VI, j=j, rb=rb):
                    V, I = VI
                    iv = candi[pl.ds(rb + q * L, L)]
                    valid = (cnts[j] - lane) > q * L
                    xv = plsc.load_gather(rowbuf, [iv])
                    x = jnp.where(valid & (iv != PAD), xv, NEG)
                    return _merge_slice(x, iv, V, I)

                V, I = lax.fori_loop(0, nslots, mbody, (V, I))
            return (V, I)

        def slow(ops):
            def mslice(ti, VI):
                V, I = VI
                x = rowbuf[pl.ds(ti * L, L)]
                iv = ti * L + lane
                x = jnp.where(iv == PAD, NEG, x)
                return _merge_slice(x, iv, V, I)

            return lax.fori_loop(0, SLICES, mslice, ops)

        vinit = (jnp.full((L,), NEG, jnp.float32), jnp.zeros((L,), jnp.int32))
        V, I = lax.cond(overflow, slow, fast, vinit)

        vstage[0] = m
        vstage[1] = s
        vstage[2] = V
        istage[...] = I
        pltpu.sync_copy(vstage, stat_hbm.at[row])
        pltpu.sync_copy(istage, idx_hbm.at[row])
        return 0

    lax.fori_loop(0, ROWS_PER_W, do_row, 0)


_sc_topk = functools.partial(
    pl.kernel,
    out_type=[
        jax.ShapeDtypeStruct((ROWS, 3, L), jnp.float32),
        jax.ShapeDtypeStruct((ROWS, L), jnp.int32),
    ],
    mesh=plsc.VectorSubcoreMesh(
        core_axis_name="c", subcore_axis_name="s",
        num_cores=NC, num_subcores=NS),
    scratch_types=[
        pltpu.VMEM((VOCAB,), jnp.float32),
        pltpu.VMEM((NCAND,), jnp.int32),
        pltpu.VMEM((3, L), jnp.float32),
        pltpu.VMEM((L,), jnp.int32),
    ],
    compiler_params=pltpu.CompilerParams(needs_layout_passes=False),
)(_sc_body)


def _tc_merge_body(m_ref, s_ref, v_ref, i_ref, sc_ref, os_ref, ot_ref, ob_ref):
    m = m_ref[...]
    s = s_ref[...]
    V = v_ref[...]
    I = i_ref[...]
    sc = sc_ref[...]

    seg = lax.broadcasted_iota(jnp.int32, (BSZ, BEAM * L), 1) // L
    offs = jnp.zeros((BSZ, BEAM * L), jnp.float32)
    for j in range(BEAM):
        maskj = seg == j
        mj = jnp.max(jnp.where(maskj, m, NEG), axis=1, keepdims=True)
        sj = jnp.sum(jnp.where(maskj, s * jnp.exp(m - mj), 0.0),
                     axis=1, keepdims=True)
        offj = sc[:, j:j + 1] - (mj + jnp.log(sj))
        offs = jnp.where(maskj, offj, offs)

    cand = V + offs
    flat = I + seg * VOCAB
    big = jnp.int32(2**31 - 1)
    vals, flats = [], []
    for _ in range(K2):
        cur = jnp.max(cand, axis=1, keepdims=True)
        cf = jnp.min(jnp.where(cand == cur, flat, big), axis=1, keepdims=True)
        vals.append(cur)
        flats.append(cf)
        cand = jnp.where(flat == cf, NEG, cand)
    ts = jnp.concatenate(vals, axis=1)
    tf = jnp.concatenate(flats, axis=1)
    os_ref[...] = ts
    ot_ref[...] = tf % VOCAB
    ob_ref[...] = tf // VOCAB


_tc_merge = pl.pallas_call(
    _tc_merge_body,
    out_shape=(
        jax.ShapeDtypeStruct((BSZ, K2), jnp.float32),
        jax.ShapeDtypeStruct((BSZ, K2), jnp.int32),
        jax.ShapeDtypeStruct((BSZ, K2), jnp.int32),
    ),
)


def kernel(logits, scores):
    stat, idx = _sc_topk(logits)
    m4 = stat[:, 0, :].reshape(BSZ, BEAM * L)
    s4 = stat[:, 1, :].reshape(BSZ, BEAM * L)
    v4 = stat[:, 2, :].reshape(BSZ, BEAM * L)
    i4 = idx.reshape(BSZ, BEAM * L)
    sc4 = scores.reshape(BSZ, BEAM)
    return _tc_merge(m4, s4, v4, i4, sc4)
